# baseline (speedup 1.0000x reference)
import functools

import jax
import jax.numpy as jnp
from jax import lax
from jax.experimental import pallas as pl
from jax.experimental.pallas import tpu as pltpu
from jax.experimental.pallas import tpu_sc as plsc

N_ENT = 10000
N_REL = 24
D = 256
E = 160000
RES_LAMBDA = 0.5
MIN_NORM = 1e-15
MAX_NORM = 1.0 - 1e-5

W = 128
IDX_ROWS = E // W
GIDX_ROWS = 2 * E // W
NC, NS = 2, 16
NW = NC * NS
HALF = D // 2
CHUNK = 624
TAIL = N_ENT - NS * CHUNK
BE = 2000
NB = E // BE
RB = 1000

@functools.lru_cache(maxsize=None)
def _get_sc_gather():
    mesh = plsc.VectorSubcoreMesh(
        core_axis_name="c", subcore_axis_name="s",
        num_cores=NC, num_subcores=NS)
    return pl.kernel(
        _sc_gather_body,
        mesh=mesh,
        out_type=jax.ShapeDtypeStruct((2 * E, D), jnp.float32),
        scratch_types=[
            pltpu.VMEM((W,), jnp.int32),
            pltpu.VMEM((W, D), jnp.float32),
            pltpu.SemaphoreType.DMA,
        ],
    )


def _sc_gather_body(table_hbm, idx_hbm, out_hbm, idx_v, rows_v, sem):
    wid = lax.axis_index("s") * NC + lax.axis_index("c")

    @pl.loop(0, (GIDX_ROWS + NW - 1) // NW)
    def _(w):
        r = wid + NW * w

        @pl.when(r < GIDX_ROWS)
        def _():
            pltpu.sync_copy(idx_hbm.at[pl.ds(r * W, W)], idx_v)
            pltpu.async_copy(table_hbm.at[idx_v], rows_v, sem).wait()
            pltpu.sync_copy(rows_v, out_hbm.at[pl.ds(r * W, W)])


@functools.lru_cache(maxsize=None)
def _get_sc_scatter():
    mesh = plsc.VectorSubcoreMesh(
        core_axis_name="c", subcore_axis_name="s",
        num_cores=NC, num_subcores=NS)
    return pl.kernel(
        _sc_scatter_body,
        mesh=mesh,
        out_type=[
            jax.ShapeDtypeStruct((N_ENT, HALF), jnp.float32),
            jax.ShapeDtypeStruct((N_ENT, HALF), jnp.float32),
        ],
        scratch_types=[
            pltpu.VMEM((W,), jnp.int32),
            pltpu.VMEM((W, HALF), jnp.float32),
            pltpu.VMEM_SHARED((N_ENT, HALF), jnp.float32),
        ],
    )


def _sc_scatter_body(idx_hbm, res_lo_hbm, res_hi_hbm, zeros_hbm,
                     out_lo_hbm, out_hi_hbm, idx_v, buf_v, acc_sh):
    c = lax.axis_index("c")
    s = lax.axis_index("s")

    pltpu.sync_copy(zeros_hbm, acc_sh.at[pl.ds(s * CHUNK, CHUNK)])

    @pl.when(s == 0)
    def _():
        pltpu.sync_copy(zeros_hbm.at[pl.ds(0, TAIL)],
                        acc_sh.at[pl.ds(NS * CHUNK, TAIL)])

    plsc.subcore_barrier()

    def accumulate(res_hbm):
        @pl.loop(0, (IDX_ROWS + NS - 1) // NS)
        def _(w):
            r = s + NS * w

            @pl.when(r < IDX_ROWS)
            def _():
                pltpu.sync_copy(idx_hbm.at[pl.ds(r * W, W)], idx_v)
                pltpu.sync_copy(res_hbm.at[pl.ds(r * W, W)], buf_v)
                pltpu.sync_copy(buf_v, acc_sh.at[idx_v], add=True)

    @pl.when(c == 0)
    def _():
        accumulate(res_lo_hbm)

    @pl.when(c == 1)
    def _():
        accumulate(res_hi_hbm)

    plsc.subcore_barrier()

    def writeout(out_hbm):
        pltpu.sync_copy(acc_sh.at[pl.ds(s * CHUNK, CHUNK)],
                        out_hbm.at[pl.ds(s * CHUNK, CHUNK)])

        @pl.when(s == 0)
        def _():
            pltpu.sync_copy(acc_sh.at[pl.ds(NS * CHUNK, TAIL)],
                            out_hbm.at[pl.ds(NS * CHUNK, TAIL)])

    @pl.when(c == 0)
    def _():
        writeout(out_lo_hbm)

    @pl.when(c == 1)
    def _():
        writeout(out_hi_hbm)


def _rsum128(r):
    acc = r[:, 0:8]
    for k in range(1, 16):
        acc = acc + r[:, 8 * k:8 * (k + 1)]
    a = acc[:, 0:4] + acc[:, 4:8]
    a = a[:, 0:2] + a[:, 2:4]
    return a[:, 0:1] + a[:, 1:2]


def _rdot(u, v):
    return _rsum128(u[:, :128] * v[:, :128] + u[:, 128:] * v[:, 128:])


def _nrm(x):
    return jnp.sqrt(jnp.maximum(_rdot(x, x), MIN_NORM))


def _project_frac(num, den, m):
    nm = jnp.sqrt(jnp.maximum(_rdot(m, m), MIN_NORM))
    return jnp.where(nm > MAX_NORM, num / (den * nm) * MAX_NORM, m)


def _mobius_frac(x, y, x2):
    y2 = _rdot(y, y)
    xy = _rdot(x, y)
    num = (2.0 * xy + 1.0 + y2) * x + (1.0 - x2) * y
    den = jnp.maximum(2.0 * xy + 1.0 + x2 * y2, MIN_NORM)
    return num, den, num / den


def _edge_math_body(gh_ref, gt_ref, rt_ref, rel_ref, lo_ref, hi_ref):
    h = gh_ref[...]
    t = gt_ref[...]
    rt = rt_ref[0, 0, :] - 1
    rel_full = rel_ref[...]
    r = jnp.where(rt[:, None] == 0, rel_full[0][None, :], 0.0)
    for k in range(1, N_REL):
        r = jnp.where(rt[:, None] == k, rel_full[k][None, :], r)

    nh = _nrm(h)
    a_num = jnp.tanh(nh) * h
    u = a_num / nh
    n = _nrm(u)
    p = jnp.where(n > MAX_NORM, a_num / (nh * n) * MAX_NORM, u)

    p2 = _rdot(p, p)
    cl = jnp.maximum(1.0 - p2, MIN_NORM)
    lam = 2.0 / cl
    lam_half = lam * 0.5

    def expmap_p(v):
        nv = _nrm(v)
        second = jnp.tanh(lam_half * nv) * v / nv
        return _project_frac(*_mobius_frac(p, second, p2))

    hyper_tail = expmap_p(t)
    hyper_rel = expmap_p(r)
    ht2 = _rdot(hyper_tail, hyper_tail)
    res = _project_frac(*_mobius_frac(hyper_tail, hyper_rel, ht2))

    _, _, sub = _mobius_frac(-p, res, p2)
    ns = _nrm(sub)
    nc = jnp.clip(ns, -1.0 + 1e-7, 1.0 - 1e-7)
    artanh = 0.5 * (jnp.log1p(nc) - jnp.log1p(-nc))
    res = (2.0 / lam) * artanh * sub / ns

    ricci = t + r
    rn = ricci / jnp.clip(jnp.sqrt(_rdot(ricci, ricci)), 1e-12, None)
    res = jax.nn.relu(res + rn * 1e-7)

    lo_ref[...] = res[:, :HALF]
    hi_ref[...] = res[:, HALF:]


def _edge_math(gathered, rtype3, relation_emb):
    return pl.pallas_call(
        _edge_math_body,
        grid=(NB,),
        in_specs=[
            pl.BlockSpec((BE, D), lambda i: (i, 0)),
            pl.BlockSpec((BE, D), lambda i: (i + NB, 0)),
            pl.BlockSpec((1, 1, BE), lambda i: (i, 0, 0)),
            pl.BlockSpec((N_REL, D), lambda i: (0, 0)),
        ],
        out_specs=[
            pl.BlockSpec((BE, HALF), lambda i: (i, 0)),
            pl.BlockSpec((BE, HALF), lambda i: (i, 0)),
        ],
        out_shape=[
            jax.ShapeDtypeStruct((E, HALF), jnp.float32),
            jax.ShapeDtypeStruct((E, HALF), jnp.float32),
        ],
    )(gathered, gathered, rtype3, relation_emb)


def _norm_body(lo_ref, hi_ref, out_ref):
    sums = jnp.concatenate([lo_ref[...], hi_ref[...]], axis=-1)
    n = jnp.sqrt(jnp.sum(sums * sums, axis=-1, keepdims=True))
    out_ref[...] = sums / jnp.clip(n, 1e-12, None)


def _normalize(lo, hi):
    return pl.pallas_call(
        _norm_body,
        grid=(N_ENT // RB,),
        in_specs=[
            pl.BlockSpec((RB, HALF), lambda i: (i, 0)),
            pl.BlockSpec((RB, HALF), lambda i: (i, 0)),
        ],
        out_specs=pl.BlockSpec((RB, D), lambda i: (i, 0)),
        out_shape=jax.ShapeDtypeStruct((N_ENT, D), jnp.float32),
    )(lo, hi)


def _final_body(lo_ref, hi_ref, n1_ref, e0_ref, out_ref):
    sums = jnp.concatenate([lo_ref[...], hi_ref[...]], axis=-1)
    n = jnp.sqrt(jnp.sum(sums * sums, axis=-1, keepdims=True))
    ent2 = sums / jnp.clip(n, 1e-12, None)
    out_ref[...] = (RES_LAMBDA * RES_LAMBDA) * e0_ref[...] + RES_LAMBDA * n1_ref[...] + ent2


def _final(lo, hi, n1, e0):
    return pl.pallas_call(
        _final_body,
        grid=(N_ENT // RB,),
        in_specs=[
            pl.BlockSpec((RB, HALF), lambda i: (i, 0)),
            pl.BlockSpec((RB, HALF), lambda i: (i, 0)),
            pl.BlockSpec((RB, D), lambda i: (i, 0)),
            pl.BlockSpec((RB, D), lambda i: (i, 0)),
        ],
        out_specs=pl.BlockSpec((RB, D), lambda i: (i, 0)),
        out_shape=jax.ShapeDtypeStruct((N_ENT, D), jnp.float32),
    )(lo, hi, n1, e0)


def kernel(entity_emb, relation_emb, edge_index, edge_type):
    head = edge_index[0]
    tail = edge_index[1]
    gidx = jnp.concatenate([head, tail])
    sidx = head
    rtype3 = edge_type.reshape(NB, 1, BE)
    zeros = jnp.zeros((CHUNK, HALF), jnp.float32)

    sc_gather = _get_sc_gather()
    sc_scatter = _get_sc_scatter()

    def hop(ent):
        gathered = sc_gather(ent, gidx)
        res_lo, res_hi = _edge_math(gathered, rtype3, relation_emb)
        return sc_scatter(sidx, res_lo, res_hi, zeros)

    lo1, hi1 = hop(entity_emb)
    n1 = _normalize(lo1, hi1)
    lo2, hi2 = hop(n1)
    return _final(lo2, hi2, n1, entity_emb)
